# baseline (speedup 1.0000x reference)
import functools

import jax
import jax.numpy as jnp
from jax import lax
from jax.experimental import pallas as pl
from jax.experimental.pallas import tpu as pltpu
from jax.experimental.pallas import tpu_sc as plsc

_NC = 2
_NS = 16
_NW = _NC * _NS
_L = 16

_CHUNK_ROWS = 16


@functools.cache
def _make_sc_add(B, S, D):
    rows = B * S
    rows_w = rows // _NW
    E = _CHUNK_ROWS * D
    nch = rows_w // _CHUNK_ROWS

    mesh = plsc.VectorSubcoreMesh(
        core_axis_name="c", subcore_axis_name="s",
        num_cores=_NC, num_subcores=_NS)

    def body(x_hbm, t_hbm, o_hbm,
             xb0, xb1, tb0, tb1, ob0, ob1,
             slx0, slx1, slt0, slt1, sst0, sst1):
        wid = lax.axis_index("s") * _NC + lax.axis_index("c")
        row0 = wid * rows_w
        xoff0 = row0 * D
        toff0 = lax.rem(row0, S) * D

        xbufs = (xb0, xb1)
        tbufs = (tb0, tb1)
        obufs = (ob0, ob1)
        slx = (slx0, slx1)
        slt = (slt0, slt1)
        sst = (sst0, sst1)

        def start_load(c, b):
            pltpu.async_copy(x_hbm.at[pl.ds(xoff0 + c * E, E)], xbufs[b], slx[b])
            pltpu.async_copy(t_hbm.at[pl.ds(toff0 + c * E, E)], tbufs[b], slt[b])

        def wait_load(b):
            pltpu.make_async_copy(x_hbm.at[pl.ds(0, E)], xbufs[b], slx[b]).wait()
            pltpu.make_async_copy(t_hbm.at[pl.ds(0, E)], tbufs[b], slt[b]).wait()

        def wait_store(b):
            pltpu.make_async_copy(obufs[b], o_hbm.at[pl.ds(0, E)], sst[b]).wait()

        start_load(0, 0)
        start_load(1, 1)

        @pl.loop(0, nch, step=2)
        def _chunks(cc):
            for b in (0, 1):
                c = cc + b

                @pl.when(c >= 2)
                def _():
                    wait_store(b)

                wait_load(b)
                xb, tb, ob = xbufs[b], tbufs[b], obufs[b]

                @plsc.parallel_loop(0, E, step=_L, unroll=8)
                def _add(i):
                    ob[pl.ds(i, _L)] = xb[pl.ds(i, _L)] + tb[pl.ds(i, _L)]

                pltpu.async_copy(ob, o_hbm.at[pl.ds(xoff0 + c * E, E)], sst[b])

                @pl.when(c + 2 < nch)
                def _():
                    start_load(c + 2, b)

        wait_store(0)
        wait_store(1)

    f32 = jnp.float32
    return pl.kernel(
        body,
        out_type=jax.ShapeDtypeStruct((rows * D,), f32),
        mesh=mesh,
        scratch_types=[
            pltpu.VMEM((E,), f32), pltpu.VMEM((E,), f32),
            pltpu.VMEM((E,), f32), pltpu.VMEM((E,), f32),
            pltpu.VMEM((E,), f32), pltpu.VMEM((E,), f32),
            pltpu.SemaphoreType.DMA, pltpu.SemaphoreType.DMA,
            pltpu.SemaphoreType.DMA, pltpu.SemaphoreType.DMA,
            pltpu.SemaphoreType.DMA, pltpu.SemaphoreType.DMA,
        ],
    )


def kernel(x, pos_emb_table):
    B, S, D = x.shape
    out = _make_sc_add(B, S, D)(x.reshape(-1), pos_emb_table.reshape(-1))
    return out.reshape(B, S, D)


# speedup vs baseline: 2.6912x; 2.6912x over previous
import functools

import jax
import jax.numpy as jnp
from jax import lax
from jax.experimental import pallas as pl
from jax.experimental.pallas import tpu as pltpu
from jax.experimental.pallas import tpu_sc as plsc

_NC = 2
_NS = 16
_NW = _NC * _NS
_L = 16

_CHUNK_ROWS = 16


@functools.cache
def _make_sc_add(B, S, D):
    rows = B * S
    rows_w = rows // _NW
    segs = S // rows_w
    C = _CHUNK_ROWS
    nch = rows_w // C
    nj = D // _L

    mesh = plsc.VectorSubcoreMesh(
        core_axis_name="c", subcore_axis_name="s",
        num_cores=_NC, num_subcores=_NS)

    def body(x_hbm, t_hbm, o_hbm,
             xb0, xb1, tb0, tb1, ob0, ob1,
             slx0, slx1, slt0, slt1, sst0, sst1):
        wid = lax.axis_index("s") * _NC + lax.axis_index("c")
        b = wid // segs
        r0 = lax.rem(wid, segs) * rows_w

        xbufs = (xb0, xb1)
        tbufs = (tb0, tb1)
        obufs = (ob0, ob1)
        slx = (slx0, slx1)
        slt = (slt0, slt1)
        sst = (sst0, sst1)

        def start_load(c, k):
            r = r0 + c * C
            pltpu.async_copy(x_hbm.at[b, pl.ds(r, C), :], xbufs[k], slx[k])
            pltpu.async_copy(t_hbm.at[pl.ds(r, C), :], tbufs[k], slt[k])

        def wait_load(k):
            pltpu.make_async_copy(x_hbm.at[b, pl.ds(0, C), :], xbufs[k], slx[k]).wait()
            pltpu.make_async_copy(t_hbm.at[pl.ds(0, C), :], tbufs[k], slt[k]).wait()

        def wait_store(k):
            pltpu.make_async_copy(obufs[k], o_hbm.at[b, pl.ds(0, C), :], sst[k]).wait()

        start_load(0, 0)
        start_load(1, 1)

        @pl.loop(0, nch, step=2)
        def _chunks(cc):
            for k in (0, 1):
                c = cc + k

                @pl.when(c >= 2)
                def _():
                    wait_store(k)

                wait_load(k)
                xb, tb, ob = xbufs[k], tbufs[k], obufs[k]

                @plsc.parallel_loop(0, C, step=1, unroll=2)
                def _add(r):
                    for j in range(nj):
                        sl = pl.ds(j * _L, _L)
                        ob[r, sl] = xb[r, sl] + tb[r, sl]

                pltpu.async_copy(ob, o_hbm.at[b, pl.ds(r0 + c * C, C), :], sst[k])

                @pl.when(c + 2 < nch)
                def _():
                    start_load(c + 2, k)

        wait_store(0)
        wait_store(1)

    f32 = jnp.float32
    return pl.kernel(
        body,
        out_type=jax.ShapeDtypeStruct((B, S, D), f32),
        mesh=mesh,
        scratch_types=[
            pltpu.VMEM((C, D), f32), pltpu.VMEM((C, D), f32),
            pltpu.VMEM((C, D), f32), pltpu.VMEM((C, D), f32),
            pltpu.VMEM((C, D), f32), pltpu.VMEM((C, D), f32),
            pltpu.SemaphoreType.DMA, pltpu.SemaphoreType.DMA,
            pltpu.SemaphoreType.DMA, pltpu.SemaphoreType.DMA,
            pltpu.SemaphoreType.DMA, pltpu.SemaphoreType.DMA,
        ],
        compiler_params=pltpu.CompilerParams(use_tc_tiling_on_sc=True),
    )


def kernel(x, pos_emb_table):
    B, S, D = x.shape
    return _make_sc_add(B, S, D)(x, pos_emb_table)


# speedup vs baseline: 2.9670x; 1.1025x over previous
import functools

import jax
import jax.numpy as jnp
from jax import lax
from jax.experimental import pallas as pl
from jax.experimental.pallas import tpu as pltpu
from jax.experimental.pallas import tpu_sc as plsc

_NC = 2
_NS = 16
_NW = _NC * _NS
_L = 16

_CHUNK_ROWS = 16


@functools.cache
def _make_sc_add(B, S, D):
    seq_w = S // _NW
    C = _CHUNK_ROWS
    nch = seq_w // C
    nit = nch * B
    nj = D // _L

    mesh = plsc.VectorSubcoreMesh(
        core_axis_name="c", subcore_axis_name="s",
        num_cores=_NC, num_subcores=_NS)

    def body(x_hbm, t_hbm, o_hbm,
             xb0, xb1, tb0, tb1, ob0, ob1,
             slx0, slx1, slt0, slt1, sst0, sst1):
        wid = lax.axis_index("s") * _NC + lax.axis_index("c")
        r0 = wid * seq_w

        xbufs = (xb0, xb1)
        tbufs = (tb0, tb1)
        obufs = (ob0, ob1)
        slx = (slx0, slx1)
        slt = (slt0, slt1)
        sst = (sst0, sst1)

        def start_load_t(c, k):
            pltpu.async_copy(t_hbm.at[pl.ds(r0 + c * C, C), :], tbufs[k], slt[k])

        def wait_load_t(k):
            pltpu.make_async_copy(t_hbm.at[pl.ds(0, C), :], tbufs[k], slt[k]).wait()

        def start_load_x(c, b, k):
            pltpu.async_copy(x_hbm.at[b, pl.ds(r0 + c * C, C), :], xbufs[k], slx[k])

        def wait_load_x(k):
            pltpu.make_async_copy(x_hbm.at[0, pl.ds(0, C), :], xbufs[k], slx[k]).wait()

        def wait_store(k):
            pltpu.make_async_copy(obufs[k], o_hbm.at[0, pl.ds(0, C), :], sst[k]).wait()

        start_load_t(0, 0)
        start_load_t(1, 1)
        start_load_x(0, 0, 0)
        start_load_x(0, 1, 1)

        @pl.loop(0, nch, step=2)
        def _chunks(c):
            for tk in (0, 1):
                cc = c + tk

                wait_load_t(tk)
                for b in range(B):
                    k = b % 2
                    if b >= 2:
                        wait_store(k)
                    else:
                        @pl.when(cc >= 1)
                        def _():
                            wait_store(k)

                    wait_load_x(k)
                    xb, tb, ob = xbufs[k], tbufs[tk], obufs[k]

                    @plsc.parallel_loop(0, C, step=1, unroll=2)
                    def _add(r):
                        for j in range(nj):
                            sl = pl.ds(j * _L, _L)
                            ob[r, sl] = xb[r, sl] + tb[r, sl]

                    pltpu.async_copy(
                        ob, o_hbm.at[b, pl.ds(r0 + cc * C, C), :], sst[k])

                    b2 = (b + 2) % B
                    cc2 = cc + (1 if b + 2 >= B else 0)

                    @pl.when(cc2 < nch)
                    def _():
                        start_load_x(cc2, b2, k)

                @pl.when(cc + 2 < nch)
                def _():
                    start_load_t(cc + 2, tk)

        wait_store(0)
        wait_store(1)

    f32 = jnp.float32
    return pl.kernel(
        body,
        out_type=jax.ShapeDtypeStruct((B, S, D), f32),
        mesh=mesh,
        scratch_types=[
            pltpu.VMEM((C, D), f32), pltpu.VMEM((C, D), f32),
            pltpu.VMEM((C, D), f32), pltpu.VMEM((C, D), f32),
            pltpu.VMEM((C, D), f32), pltpu.VMEM((C, D), f32),
            pltpu.SemaphoreType.DMA, pltpu.SemaphoreType.DMA,
            pltpu.SemaphoreType.DMA, pltpu.SemaphoreType.DMA,
            pltpu.SemaphoreType.DMA, pltpu.SemaphoreType.DMA,
        ],
        compiler_params=pltpu.CompilerParams(use_tc_tiling_on_sc=True),
    )


def kernel(x, pos_emb_table):
    B, S, D = x.shape
    return _make_sc_add(B, S, D)(x, pos_emb_table)


# speedup vs baseline: 3.3541x; 1.1304x over previous
import functools

import jax
import jax.numpy as jnp
from jax import lax
from jax.experimental import pallas as pl
from jax.experimental.pallas import tpu as pltpu
from jax.experimental.pallas import tpu_sc as plsc

_NC = 2
_NS = 16
_NW = _NC * _NS
_L = 16

_CHUNK_ROWS = 16


@functools.cache
def _make_sc_add(B, S, D):
    seq_w = S // _NW
    C = _CHUNK_ROWS
    nch = seq_w // C
    nj = D // _L
    npair = B // 2

    mesh = plsc.VectorSubcoreMesh(
        core_axis_name="c", subcore_axis_name="s",
        num_cores=_NC, num_subcores=_NS)

    def body(x_hbm, t_hbm, o_hbm,
             xb0, xb1, xb2, xb3, ob0, ob1, ob2, ob3, tb0, tb1,
             slx0, slx1, slx2, slx3, sst0, sst1, sst2, sst3, slt0, slt1):
        wid = lax.axis_index("s") * _NC + lax.axis_index("c")
        r0 = wid * seq_w

        xbufs = (xb0, xb1, xb2, xb3)
        obufs = (ob0, ob1, ob2, ob3)
        tbufs = (tb0, tb1)
        slx = (slx0, slx1, slx2, slx3)
        sst = (sst0, sst1, sst2, sst3)
        slt = (slt0, slt1)

        def start_load_t(c, k):
            pltpu.async_copy(t_hbm.at[pl.ds(r0 + c * C, C), :], tbufs[k], slt[k])

        def wait_load_t(k):
            pltpu.make_async_copy(t_hbm.at[pl.ds(0, C), :], tbufs[k], slt[k]).wait()

        def start_load_x(c, b, k):
            pltpu.async_copy(x_hbm.at[b, pl.ds(r0 + c * C, C), :], xbufs[k], slx[k])

        def wait_load_x(k):
            pltpu.make_async_copy(x_hbm.at[0, pl.ds(0, C), :], xbufs[k], slx[k]).wait()

        def wait_store(k):
            pltpu.make_async_copy(obufs[k], o_hbm.at[0, pl.ds(0, C), :], sst[k]).wait()

        start_load_t(0, 0)
        start_load_t(1, 1)
        for b in range(B):
            start_load_x(0, b, b)

        @pl.loop(0, nch, step=2)
        def _chunks(c):
            for tk in (0, 1):
                cc = c + tk
                wait_load_t(tk)
                for p in range(npair):
                    b0, b1 = 2 * p, 2 * p + 1
                    k0, k1 = 2 * p, 2 * p + 1

                    @pl.when(cc >= 1)
                    def _():
                        wait_store(k0)
                        wait_store(k1)

                    wait_load_x(k0)
                    wait_load_x(k1)
                    xa, xc = xbufs[k0], xbufs[k1]
                    oa, oc = obufs[k0], obufs[k1]
                    tb = tbufs[tk]

                    @plsc.parallel_loop(0, C, step=1, unroll=2)
                    def _add(r):
                        for j in range(nj):
                            sl = pl.ds(j * _L, _L)
                            vt = tb[r, sl]
                            oa[r, sl] = xa[r, sl] + vt
                            oc[r, sl] = xc[r, sl] + vt

                    pltpu.async_copy(
                        oa, o_hbm.at[b0, pl.ds(r0 + cc * C, C), :], sst[k0])
                    pltpu.async_copy(
                        oc, o_hbm.at[b1, pl.ds(r0 + cc * C, C), :], sst[k1])

                    @pl.when(cc + 1 < nch)
                    def _():
                        start_load_x(cc + 1, b0, k0)
                        start_load_x(cc + 1, b1, k1)

                @pl.when(cc + 2 < nch)
                def _():
                    start_load_t(cc + 2, tk)

        for k in range(2 * npair):
            wait_store(k)

    f32 = jnp.float32
    return pl.kernel(
        body,
        out_type=jax.ShapeDtypeStruct((B, S, D), f32),
        mesh=mesh,
        scratch_types=(
            [pltpu.VMEM((C, D), f32)] * 10
            + [pltpu.SemaphoreType.DMA] * 10
        ),
        compiler_params=pltpu.CompilerParams(use_tc_tiling_on_sc=True),
    )


def kernel(x, pos_emb_table):
    B, S, D = x.shape
    return _make_sc_add(B, S, D)(x, pos_emb_table)
